# baseline (speedup 1.0000x reference)
import jax
import jax.numpy as jnp
from jax import lax
from jax.experimental import pallas as pl
from jax.experimental.pallas import tpu as pltpu
from jax.experimental.pallas import tpu_sc as plsc

N = 10000
N2 = 2 * N
E = 320000
D = 128
DH = D // 2
NC = 2
NS = 16
NW = NC * NS
EW = E // NW
G = 80
NG = EW // G
NP = 10240
NP2 = 2 * NP
RPT = NP2 // NS
LANES = 16
NBUF = 4
G2 = 80
NG2 = EW // G2
NB2 = 5


def _prep1_body(x_ref, w_ref, al_ref, ar_ref, wh_ref, el_ref, er_ref):
    wh = jnp.dot(x_ref[...], w_ref[...], preferred_element_type=jnp.float32)
    wh_ref[...] = wh
    el_ref[...] = jnp.dot(wh, al_ref[...], preferred_element_type=jnp.float32)
    er_ref[...] = jnp.dot(wh, ar_ref[...], preferred_element_type=jnp.float32)


def _prep1(x, W, a_left, a_right):
    blk = N // 10
    return pl.pallas_call(
        _prep1_body,
        grid=(10,),
        in_specs=[
            pl.BlockSpec((blk, D), lambda i: (i, 0)),
            pl.BlockSpec((D, D), lambda i: (0, 0)),
            pl.BlockSpec((D, 1), lambda i: (0, 0)),
            pl.BlockSpec((D, 1), lambda i: (0, 0)),
        ],
        out_specs=[
            pl.BlockSpec((blk, D), lambda i: (i, 0)),
            pl.BlockSpec((blk, 1), lambda i: (i, 0)),
            pl.BlockSpec((blk, 1), lambda i: (i, 0)),
        ],
        out_shape=[
            jax.ShapeDtypeStruct((N, D), jnp.float32),
            jax.ShapeDtypeStruct((N, 1), jnp.float32),
            jax.ShapeDtypeStruct((N, 1), jnp.float32),
        ],
    )(x, W, a_left, a_right)


def _prep2_body(wh_ref, er_ref, y_ref):
    i = pl.program_id(0)
    wh = wh_ref[...]
    f = jnp.where(i < 10, jnp.exp(er_ref[...]), jnp.exp(0.2 * er_ref[...]))
    y_ref[0] = wh[:, :DH] * f
    y_ref[1] = wh[:, DH:] * f


def _prep2(wh, er):
    blk = N // 10
    return pl.pallas_call(
        _prep2_body,
        grid=(20,),
        in_specs=[
            pl.BlockSpec((blk, D), lambda i: (i % 10, 0)),
            pl.BlockSpec((blk, 1), lambda i: (i % 10, 0)),
        ],
        out_specs=pl.BlockSpec((2, blk, DH), lambda i: (0, i, 0)),
        out_shape=jax.ShapeDtypeStruct((2, N2, DH), jnp.float32),
    )(wh, er)


def _sc1_body(ei_hbm, el_hbm, er_hbm,
              i2_hbm, i3_hbm, psum_hbm,
              src_v, dst_v, el_v, er_v,
              vb0, vb1, vb2, vb3, psh,
              semi, sp0, sp1, sp2, sp3):
    vbufs = [vb0, vb1, vb2, vb3]
    semp = [sp0, sp1, sp2, sp3]

    cid = lax.axis_index("c")
    sid = lax.axis_index("s")
    wid = cid * NS + sid

    cps = [
        pltpu.async_copy(ei_hbm.at[0, wid], src_v, semi),
        pltpu.async_copy(ei_hbm.at[1, wid], dst_v, semi),
        pltpu.async_copy(el_hbm, el_v, semi),
        pltpu.async_copy(er_hbm, er_v, semi),
    ]

    zf = jnp.zeros((LANES,), jnp.float32)

    def _zero_vbufs(g, _):
        for vb in vbufs:
            vb[g] = zf
        return 0

    lax.fori_loop(0, G, _zero_vbufs, 0)
    for k in range(RPT // G):
        pltpu.sync_copy(vb0, psh.at[pl.ds(sid * RPT + k * G, G)])
    for cp in cps:
        cp.wait()
    plsc.subcore_barrier()

    lane_iota = lax.iota(jnp.int32, LANES)
    zi = jnp.zeros((LANES,), jnp.int32)

    def _compute_group(gi, vb):
        for w in range(G // LANES):
            sl = pl.ds(w * LANES, LANES)
            isrc = src_v[gi, sl]
            idst = dst_v[gi, sl]
            elg = plsc.load_gather(el_v, [isrc])
            erg = plsc.load_gather(er_v, [idst])
            pos = (elg + erg) > 0
            v = jnp.exp(jnp.where(pos, erg, 0.2 * erg))
            src_v[gi, sl] = jnp.where(pos, isrc, isrc + NP)
            dst_v[gi, sl] = jnp.where(pos, idst, idst + N)
            plsc.store_scatter(vb, [w * LANES + lane_iota, zi], v)

    _compute_group(0, vb0)
    pltpu.sync_copy(vb0, psh.at[src_v.at[0]], add=True)

    def _p1_step(i, _):
        for j in range(NBUF):
            g = 1 + i * NBUF + j
            k = (1 + j) % NBUF

            @pl.when(g >= 5)
            def _():
                pltpu.make_async_copy(
                    vbufs[k], psh.at[src_v.at[0]], semp[k]).wait()

            _compute_group(g, vbufs[k])
            pltpu.async_copy(vbufs[k], psh.at[src_v.at[g]], semp[k], add=True)
        return 0

    lax.fori_loop(0, (NG - 1) // NBUF, _p1_step, 0)
    for k in range(NBUF):
        pltpu.make_async_copy(vbufs[k], psh.at[src_v.at[0]], semp[k]).wait()
    plsc.subcore_barrier()

    pltpu.sync_copy(psh.at[pl.ds(sid * RPT, RPT)],
                    psum_hbm.at[cid, pl.ds(sid * RPT, RPT)])
    pltpu.sync_copy(dst_v, i2_hbm.at[wid])
    pltpu.sync_copy(src_v, i3_hbm.at[wid])


_sc1_call = pl.kernel(
    _sc1_body,
    out_type=[
        jax.ShapeDtypeStruct((NW, NG, G), jnp.int32),
        jax.ShapeDtypeStruct((NW, NG, G), jnp.int32),
        jax.ShapeDtypeStruct((NC, NP2, LANES), jnp.float32),
    ],
    mesh=plsc.VectorSubcoreMesh(core_axis_name="c", subcore_axis_name="s"),
    compiler_params=pltpu.CompilerParams(
        needs_layout_passes=False, use_tc_tiling_on_sc=False),
    scratch_types=(
        [
            pltpu.VMEM((NG, G), jnp.int32),
            pltpu.VMEM((NG, G), jnp.int32),
            pltpu.VMEM((N,), jnp.float32),
            pltpu.VMEM((N,), jnp.float32),
        ]
        + [pltpu.VMEM((G, LANES), jnp.float32)] * NBUF
        + [pltpu.VMEM_SHARED((NP2, LANES), jnp.float32)]
        + [pltpu.SemaphoreType.DMA] * 5
    ),
)


def _sc2_body(i2_hbm, i3_hbm, ycat_hbm,
              pacc_hbm,
              i2_v, i3_v, r0, r1, r2, r3, r4, acc,
              semi, sg0, sg1, sg2, sg3, sg4, ss0, ss1, ss2, ss3, ss4):
    rows = [r0, r1, r2, r3, r4]
    semg = [sg0, sg1, sg2, sg3, sg4]
    sems = [ss0, ss1, ss2, ss3, ss4]

    cid = lax.axis_index("c")
    sid = lax.axis_index("s")
    y_hbm = ycat_hbm.at[cid]

    zf = jnp.zeros((LANES,), jnp.float32)

    def _zero_r0(g, _):
        for j in range(DH // LANES):
            r0[g, pl.ds(j * LANES, LANES)] = zf
        return 0

    def _grp(ref, g):
        return ref.at[pl.ds(g * G2, G2)]

    lax.fori_loop(0, G2, _zero_r0, 0)
    for k in range(RPT // G):
        pltpu.sync_copy(r0.at[pl.ds(0, G)],
                        acc.at[pl.ds(sid * RPT + k * G, G)])
    plsc.subcore_barrier()

    for chunk in range(2):
        wid = 2 * sid + chunk
        cps = [
            pltpu.async_copy(i2_hbm.at[wid], i2_v, semi),
            pltpu.async_copy(i3_hbm.at[wid], i3_v, semi),
        ]
        for cp in cps:
            cp.wait()

        pltpu.async_copy(y_hbm.at[_grp(i2_v, 0)], r0, semg[0]).wait()
        pltpu.sync_copy(r0, acc.at[_grp(i3_v, 0)], add=True)
        for g in range(1, NB2):
            pltpu.async_copy(y_hbm.at[_grp(i2_v, g)], rows[g % NB2], semg[g % NB2])

        def _p2_body(g, k, kn, gn):
            pltpu.make_async_copy(
                y_hbm.at[_grp(i2_v, 0)], rows[k], semg[k]).wait()
            pltpu.async_copy(rows[k], acc.at[_grp(i3_v, g)], sems[k],
                             add=True)

            @pl.when(gn <= NG2 - 1)
            def _():
                @pl.when(g >= 2)
                def _():
                    pltpu.make_async_copy(
                        rows[kn], acc.at[_grp(i3_v, 0)], sems[kn]).wait()

                pltpu.async_copy(y_hbm.at[_grp(i2_v, gn)], rows[kn],
                                 semg[kn])

        def _p2_step(i, _):
            for j in range(NB2):
                g = 1 + i * NB2 + j
                k = (1 + j) % NB2
                kn = (k + NB2 - 1) % NB2
                _p2_body(g, k, kn, g + NB2 - 1)
            return 0

        nfull = (NG2 - 1) // NB2
        lax.fori_loop(0, nfull, _p2_step, 0)
        for g in range(1 + nfull * NB2, NG2):
            k = g % NB2
            kn = (g + NB2 - 1) % NB2
            gn = g + NB2 - 1
            pltpu.make_async_copy(
                y_hbm.at[_grp(i2_v, 0)], rows[k], semg[k]).wait()
            pltpu.async_copy(rows[k], acc.at[_grp(i3_v, g)], sems[k],
                             add=True)
            if gn <= NG2 - 1:
                pltpu.make_async_copy(
                    rows[kn], acc.at[_grp(i3_v, 0)], sems[kn]).wait()
                pltpu.async_copy(y_hbm.at[_grp(i2_v, gn)], rows[kn],
                                 semg[kn])
        for k in range(NB2):
            pltpu.make_async_copy(
                rows[k], acc.at[_grp(i3_v, 0)], sems[k]).wait()

    plsc.subcore_barrier()
    pltpu.sync_copy(acc.at[pl.ds(sid * RPT, RPT)],
                    pacc_hbm.at[cid, pl.ds(sid * RPT, RPT)])


_sc2_call = pl.kernel(
    _sc2_body,
    out_type=[
        jax.ShapeDtypeStruct((NC, NP2, DH), jnp.float32),
    ],
    mesh=plsc.VectorSubcoreMesh(core_axis_name="c", subcore_axis_name="s"),
    compiler_params=pltpu.CompilerParams(
        needs_layout_passes=False, use_tc_tiling_on_sc=False),
    scratch_types=(
        [
            pltpu.VMEM((EW,), jnp.int32),
            pltpu.VMEM((EW,), jnp.int32),
        ]
        + [pltpu.VMEM((G2, DH), jnp.float32)] * NB2
        + [pltpu.VMEM_SHARED((NP2, DH), jnp.float32)]
        + [pltpu.SemaphoreType.DMA] * (1 + 2 * NB2)
    ),
)


def _fin_body(pacc_ref, ps_ref, el_ref, out_ref):
    el = el_ref[...]
    A = jnp.exp(el)
    A2 = jnp.exp(0.2 * el)
    ps = ps_ref[0] + ps_ref[1]
    SB = jnp.sum(ps[0], axis=1)
    SB2 = jnp.sum(ps[1], axis=1)
    Z = A[:, 0] * SB + A2[:, 0] * SB2
    inv = 1.0 / (Z + 1e-9)
    pa = pacc_ref[...]
    U0 = A * pa[0, 0] + A2 * pa[0, 1]
    U1 = A * pa[1, 0] + A2 * pa[1, 1]
    out_ref[:, :DH] = jnp.maximum(U0 * inv[:, None], 0.0)
    out_ref[:, DH:] = jnp.maximum(U1 * inv[:, None], 0.0)


def _finalize(pacc, psum, elp):
    blk = NP // 8
    return pl.pallas_call(
        _fin_body,
        grid=(8,),
        in_specs=[
            pl.BlockSpec((NC, 2, blk, DH), lambda i: (0, 0, i, 0)),
            pl.BlockSpec((NC, 2, blk, LANES), lambda i: (0, 0, i, 0)),
            pl.BlockSpec((blk, 1), lambda i: (i, 0)),
        ],
        out_specs=pl.BlockSpec((blk, D), lambda i: (i, 0)),
        out_shape=jax.ShapeDtypeStruct((NP, D), jnp.float32),
    )(pacc, psum, elp)


def kernel(x, edge_index, W, a_left, a_right):
    wh, el, er = _prep1(x, W, a_left, a_right)
    ei2 = edge_index.reshape(2, NW, NG, G)
    i2, i3, psum = _sc1_call(ei2, el.reshape(N), er.reshape(N))
    ycat = _prep2(wh, er)
    pacc, = _sc2_call(i2.reshape(NW, EW), i3.reshape(NW, EW), ycat)
    elp = jnp.pad(el, ((0, NP - N), (0, 0)))
    out = _finalize(pacc.reshape(NC, 2, NP, DH),
                    psum.reshape(NC, 2, NP, LANES), elp)
    return out[:N]


# speedup vs baseline: 1.0153x; 1.0153x over previous
import jax
import jax.numpy as jnp
from jax import lax
from jax.experimental import pallas as pl
from jax.experimental.pallas import tpu as pltpu
from jax.experimental.pallas import tpu_sc as plsc

N = 10000
N2 = 2 * N
E = 320000
D = 128
DH = D // 2
NC = 2
NS = 16
NW = NC * NS
EW = E // NW
G = 80
NG = EW // G
NP = 10240
NP2 = 2 * NP
RPT = NP2 // NS
LANES = 16
NBUF = 4
G2 = 80
NG2 = EW // G2
NB2 = 5


def _prep_body(x_ref, w_ref, al_ref, ar_ref,
               y_ref, el_ref, er_ref):
    i = pl.program_id(0)
    wh = jnp.dot(x_ref[...], w_ref[...], preferred_element_type=jnp.float32)
    el = jnp.dot(wh, al_ref[...], preferred_element_type=jnp.float32)
    er = jnp.dot(wh, ar_ref[...], preferred_element_type=jnp.float32)
    f = jnp.where(i < 10, jnp.exp(er), jnp.exp(0.2 * er))
    y_ref[0] = wh[:, :DH] * f
    y_ref[1] = wh[:, DH:] * f
    el_ref[...] = el
    er_ref[...] = er


def _prep(x, W, a_left, a_right):
    blk = N // 10
    return pl.pallas_call(
        _prep_body,
        grid=(20,),
        in_specs=[
            pl.BlockSpec((blk, D), lambda i: (i % 10, 0)),
            pl.BlockSpec((D, D), lambda i: (0, 0)),
            pl.BlockSpec((D, 1), lambda i: (0, 0)),
            pl.BlockSpec((D, 1), lambda i: (0, 0)),
        ],
        out_specs=[
            pl.BlockSpec((2, blk, DH), lambda i: (0, i, 0)),
            pl.BlockSpec((blk, 1), lambda i: (i % 10, 0)),
            pl.BlockSpec((blk, 1), lambda i: (i % 10, 0)),
        ],
        out_shape=[
            jax.ShapeDtypeStruct((2, N2, DH), jnp.float32),
            jax.ShapeDtypeStruct((N, 1), jnp.float32),
            jax.ShapeDtypeStruct((N, 1), jnp.float32),
        ],
    )(x, W, a_left, a_right)


def _sc1_body(ei_hbm, el_hbm, er_hbm,
              i2_hbm, i3_hbm, psum_hbm,
              src_v, dst_v, el_v, er_v,
              vb0, vb1, vb2, vb3, psh,
              semi, sp0, sp1, sp2, sp3):
    vbufs = [vb0, vb1, vb2, vb3]
    semp = [sp0, sp1, sp2, sp3]

    cid = lax.axis_index("c")
    sid = lax.axis_index("s")
    wid = cid * NS + sid

    cps = [
        pltpu.async_copy(ei_hbm.at[0, wid], src_v, semi),
        pltpu.async_copy(ei_hbm.at[1, wid], dst_v, semi),
        pltpu.async_copy(el_hbm, el_v, semi),
        pltpu.async_copy(er_hbm, er_v, semi),
    ]

    zf = jnp.zeros((LANES,), jnp.float32)

    def _zero_vbufs(g, _):
        for vb in vbufs:
            vb[g] = zf
        return 0

    lax.fori_loop(0, G, _zero_vbufs, 0)
    for k in range(RPT // G):
        pltpu.sync_copy(vb0, psh.at[pl.ds(sid * RPT + k * G, G)])
    for cp in cps:
        cp.wait()
    plsc.subcore_barrier()

    lane_iota = lax.iota(jnp.int32, LANES)
    zi = jnp.zeros((LANES,), jnp.int32)

    def _compute_group(gi, vb):
        for w in range(G // LANES):
            sl = pl.ds(w * LANES, LANES)
            isrc = src_v[gi, sl]
            idst = dst_v[gi, sl]
            elg = plsc.load_gather(el_v, [isrc])
            erg = plsc.load_gather(er_v, [idst])
            pos = (elg + erg) > 0
            v = jnp.exp(jnp.where(pos, erg, 0.2 * erg))
            src_v[gi, sl] = jnp.where(pos, isrc, isrc + NP)
            dst_v[gi, sl] = jnp.where(pos, idst, idst + N)
            plsc.store_scatter(vb, [w * LANES + lane_iota, zi], v)

    _compute_group(0, vb0)
    pltpu.sync_copy(vb0, psh.at[src_v.at[0]], add=True)

    def _p1_step(i, _):
        for j in range(NBUF):
            g = 1 + i * NBUF + j
            k = (1 + j) % NBUF

            @pl.when(g >= 5)
            def _():
                pltpu.make_async_copy(
                    vbufs[k], psh.at[src_v.at[0]], semp[k]).wait()

            _compute_group(g, vbufs[k])
            pltpu.async_copy(vbufs[k], psh.at[src_v.at[g]], semp[k], add=True)
        return 0

    lax.fori_loop(0, (NG - 1) // NBUF, _p1_step, 0)
    for k in range(NBUF):
        pltpu.make_async_copy(vbufs[k], psh.at[src_v.at[0]], semp[k]).wait()
    plsc.subcore_barrier()

    pltpu.sync_copy(psh.at[pl.ds(sid * RPT, RPT)],
                    psum_hbm.at[cid, pl.ds(sid * RPT, RPT)])
    pltpu.sync_copy(dst_v, i2_hbm.at[wid])
    pltpu.sync_copy(src_v, i3_hbm.at[wid])


_sc1_call = pl.kernel(
    _sc1_body,
    out_type=[
        jax.ShapeDtypeStruct((NW, NG, G), jnp.int32),
        jax.ShapeDtypeStruct((NW, NG, G), jnp.int32),
        jax.ShapeDtypeStruct((NC, NP2, LANES), jnp.float32),
    ],
    mesh=plsc.VectorSubcoreMesh(core_axis_name="c", subcore_axis_name="s"),
    compiler_params=pltpu.CompilerParams(
        needs_layout_passes=False, use_tc_tiling_on_sc=False),
    scratch_types=(
        [
            pltpu.VMEM((NG, G), jnp.int32),
            pltpu.VMEM((NG, G), jnp.int32),
            pltpu.VMEM((N,), jnp.float32),
            pltpu.VMEM((N,), jnp.float32),
        ]
        + [pltpu.VMEM((G, LANES), jnp.float32)] * NBUF
        + [pltpu.VMEM_SHARED((NP2, LANES), jnp.float32)]
        + [pltpu.SemaphoreType.DMA] * 5
    ),
)


def _sc2_body(i2_hbm, i3_hbm, ycat_hbm,
              pacc_hbm,
              i2_v, i3_v, r0, r1, r2, r3, r4, acc,
              semi, sg0, sg1, sg2, sg3, sg4, ss0, ss1, ss2, ss3, ss4):
    rows = [r0, r1, r2, r3, r4]
    semg = [sg0, sg1, sg2, sg3, sg4]
    sems = [ss0, ss1, ss2, ss3, ss4]

    cid = lax.axis_index("c")
    sid = lax.axis_index("s")
    y_hbm = ycat_hbm.at[cid]

    zf = jnp.zeros((LANES,), jnp.float32)

    def _zero_r0(g, _):
        for j in range(DH // LANES):
            r0[g, pl.ds(j * LANES, LANES)] = zf
        return 0

    def _grp(ref, g):
        return ref.at[pl.ds(g * G2, G2)]

    lax.fori_loop(0, G2, _zero_r0, 0)
    for k in range(RPT // G):
        pltpu.sync_copy(r0.at[pl.ds(0, G)],
                        acc.at[pl.ds(sid * RPT + k * G, G)])
    plsc.subcore_barrier()

    for chunk in range(2):
        wid = 2 * sid + chunk
        cps = [
            pltpu.async_copy(i2_hbm.at[wid], i2_v, semi),
            pltpu.async_copy(i3_hbm.at[wid], i3_v, semi),
        ]
        for cp in cps:
            cp.wait()

        pltpu.async_copy(y_hbm.at[_grp(i2_v, 0)], r0, semg[0]).wait()
        pltpu.sync_copy(r0, acc.at[_grp(i3_v, 0)], add=True)
        for g in range(1, NB2):
            pltpu.async_copy(y_hbm.at[_grp(i2_v, g)], rows[g % NB2], semg[g % NB2])

        def _p2_body(g, k, kn, gn):
            pltpu.make_async_copy(
                y_hbm.at[_grp(i2_v, 0)], rows[k], semg[k]).wait()
            pltpu.async_copy(rows[k], acc.at[_grp(i3_v, g)], sems[k],
                             add=True)

            @pl.when(gn <= NG2 - 1)
            def _():
                @pl.when(g >= 2)
                def _():
                    pltpu.make_async_copy(
                        rows[kn], acc.at[_grp(i3_v, 0)], sems[kn]).wait()

                pltpu.async_copy(y_hbm.at[_grp(i2_v, gn)], rows[kn],
                                 semg[kn])

        def _p2_step(i, _):
            for j in range(NB2):
                g = 1 + i * NB2 + j
                k = (1 + j) % NB2
                kn = (k + NB2 - 1) % NB2
                _p2_body(g, k, kn, g + NB2 - 1)
            return 0

        nfull = (NG2 - 1) // NB2
        lax.fori_loop(0, nfull, _p2_step, 0)
        for g in range(1 + nfull * NB2, NG2):
            k = g % NB2
            kn = (g + NB2 - 1) % NB2
            gn = g + NB2 - 1
            pltpu.make_async_copy(
                y_hbm.at[_grp(i2_v, 0)], rows[k], semg[k]).wait()
            pltpu.async_copy(rows[k], acc.at[_grp(i3_v, g)], sems[k],
                             add=True)
            if gn <= NG2 - 1:
                pltpu.make_async_copy(
                    rows[kn], acc.at[_grp(i3_v, 0)], sems[kn]).wait()
                pltpu.async_copy(y_hbm.at[_grp(i2_v, gn)], rows[kn],
                                 semg[kn])
        for k in range(NB2):
            pltpu.make_async_copy(
                rows[k], acc.at[_grp(i3_v, 0)], sems[k]).wait()

    plsc.subcore_barrier()
    pltpu.sync_copy(acc.at[pl.ds(sid * RPT, RPT)],
                    pacc_hbm.at[cid, pl.ds(sid * RPT, RPT)])


_sc2_call = pl.kernel(
    _sc2_body,
    out_type=[
        jax.ShapeDtypeStruct((NC, NP2, DH), jnp.float32),
    ],
    mesh=plsc.VectorSubcoreMesh(core_axis_name="c", subcore_axis_name="s"),
    compiler_params=pltpu.CompilerParams(
        needs_layout_passes=False, use_tc_tiling_on_sc=False),
    scratch_types=(
        [
            pltpu.VMEM((EW,), jnp.int32),
            pltpu.VMEM((EW,), jnp.int32),
        ]
        + [pltpu.VMEM((G2, DH), jnp.float32)] * NB2
        + [pltpu.VMEM_SHARED((NP2, DH), jnp.float32)]
        + [pltpu.SemaphoreType.DMA] * (1 + 2 * NB2)
    ),
)


def _fin_body(pacc_ref, ps_ref, el_ref, out_ref):
    el = el_ref[...]
    A = jnp.exp(el)
    A2 = jnp.exp(0.2 * el)
    ps = ps_ref[0] + ps_ref[1]
    SB = jnp.sum(ps[0], axis=1)
    SB2 = jnp.sum(ps[1], axis=1)
    Z = A[:, 0] * SB + A2[:, 0] * SB2
    inv = 1.0 / (Z + 1e-9)
    pa = pacc_ref[...]
    U0 = A * pa[0, 0] + A2 * pa[0, 1]
    U1 = A * pa[1, 0] + A2 * pa[1, 1]
    out_ref[:, :DH] = jnp.maximum(U0 * inv[:, None], 0.0)
    out_ref[:, DH:] = jnp.maximum(U1 * inv[:, None], 0.0)


def _finalize(pacc, psum, elp):
    blk = NP // 8
    return pl.pallas_call(
        _fin_body,
        grid=(8,),
        in_specs=[
            pl.BlockSpec((NC, 2, blk, DH), lambda i: (0, 0, i, 0)),
            pl.BlockSpec((NC, 2, blk, LANES), lambda i: (0, 0, i, 0)),
            pl.BlockSpec((blk, 1), lambda i: (i, 0)),
        ],
        out_specs=pl.BlockSpec((blk, D), lambda i: (i, 0)),
        out_shape=jax.ShapeDtypeStruct((NP, D), jnp.float32),
    )(pacc, psum, elp)


def kernel(x, edge_index, W, a_left, a_right):
    ycat, el, er = _prep(x, W, a_left, a_right)
    ei2 = edge_index.reshape(2, NW, NG, G)
    i2, i3, psum = _sc1_call(ei2, el.reshape(N), er.reshape(N))
    pacc, = _sc2_call(i2.reshape(NW, EW), i3.reshape(NW, EW), ycat)
    elp = jnp.pad(el, ((0, NP - N), (0, 0)))
    out = _finalize(pacc.reshape(NC, 2, NP, DH),
                    psum.reshape(NC, 2, NP, LANES), elp)
    return out[:N]
